# Initial kernel scaffold; baseline (speedup 1.0000x reference)
import jax
import jax.numpy as jnp
from jax.experimental import pallas as pl


def kernel(x, edge_index, W1, b1, W2, b2, Wf, bf):
    raise NotImplementedError("write your pallas kernel here")


# speedup vs baseline: 3.7923x; 3.7923x over previous
import functools

import jax
import jax.numpy as jnp
from jax import lax
from jax.experimental import pallas as pl
from jax.experimental.pallas import tpu as pltpu
from jax.experimental.pallas import tpu_sc as plsc

_N = 10000
_E = 320000
_DIN = 128
_HID = 256
_NPAD = 10240
_EPAD = 323584
_NSUB = 16
_NCORE = 2
_K = 128
_ROWS_PER_SUB = _NPAD // _NSUB
_ZROWS = 128
_BM = 1024
_GRID = _NPAD // _BM

_vec_mesh = plsc.VectorSubcoreMesh(core_axis_name="c", subcore_axis_name="s")


def _zero_fill_1d(buf, n):
    @pl.loop(0, n // 16)
    def _(i):
        buf[pl.ds(i * 16, 16)] = jnp.zeros((16,), jnp.float32)


def _zero_fill_2d(buf, rows, cols):
    @pl.loop(0, rows)
    def _(r):
        @pl.loop(0, cols // 16)
        def _(j):
            buf[r, pl.ds(j * 16, 16)] = jnp.zeros((16,), jnp.float32)


def _deg_call(ei):
    eps = _EPAD // _NSUB
    nfull = eps // _K

    @functools.partial(
        pl.kernel,
        out_type=jax.ShapeDtypeStruct((_NCORE, _NPAD), jnp.float32),
        mesh=_vec_mesh,
        scratch_types=[
            pltpu.VMEM((_K,), jnp.int32),
            pltpu.VMEM((_K,), jnp.float32),
            pltpu.VMEM((_ROWS_PER_SUB,), jnp.float32),
            pltpu.VMEM_SHARED((_NPAD,), jnp.float32),
            pltpu.SemaphoreType.DMA,
        ],
    )
    def k(ei_hbm, out_hbm, idx_b, ones_b, zb, acc, sem):
        c = lax.axis_index("c")
        s = lax.axis_index("s")
        _zero_fill_1d(zb, _ROWS_PER_SUB)

        @pl.loop(0, _K // 16)
        def _(i):
            ones_b[pl.ds(i * 16, 16)] = jnp.ones((16,), jnp.float32)

        pltpu.sync_copy(zb, acc.at[pl.ds(s * _ROWS_PER_SUB, _ROWS_PER_SUB)])
        plsc.subcore_barrier()

        ebase = s * eps

        @pl.loop(0, nfull)
        def _(i):
            pltpu.sync_copy(ei_hbm.at[c].at[pl.ds(ebase + i * _K, _K)], idx_b)
            pltpu.sync_copy(ones_b, acc.at[idx_b], add=True)

        plsc.subcore_barrier()
        pltpu.sync_copy(
            acc.at[pl.ds(s * _ROWS_PER_SUB, _ROWS_PER_SUB)],
            out_hbm.at[c].at[pl.ds(s * _ROWS_PER_SUB, _ROWS_PER_SUB)],
        )

    return k(ei)


def _agg_edge_split_call(ei, table):
    epc = _EPAD // _NCORE
    eps = epc // _NSUB
    nfull = eps // _K

    @functools.partial(
        pl.kernel,
        out_type=jax.ShapeDtypeStruct((_NCORE, _NPAD, _DIN), jnp.float32),
        mesh=_vec_mesh,
        scratch_types=[
            pltpu.VMEM((_K,), jnp.int32),
            pltpu.VMEM((_K,), jnp.int32),
            pltpu.VMEM((_K, _DIN), jnp.float32),
            pltpu.VMEM((_ZROWS, _DIN), jnp.float32),
            pltpu.VMEM_SHARED((_NPAD, _DIN), jnp.float32),
            pltpu.SemaphoreType.DMA,
        ],
    )
    def k(ei_hbm, tab_hbm, out_hbm, src_b, dst_b, rows_b, zb, acc, sem):
        c = lax.axis_index("c")
        s = lax.axis_index("s")
        _zero_fill_2d(zb, _ZROWS, _DIN)

        @pl.loop(0, _ROWS_PER_SUB // _ZROWS)
        def _(i):
            pltpu.sync_copy(
                zb, acc.at[pl.ds(s * _ROWS_PER_SUB + i * _ZROWS, _ZROWS)])

        plsc.subcore_barrier()

        ebase = c * epc + s * eps

        @pl.loop(0, nfull)
        def _(i):
            pltpu.sync_copy(ei_hbm.at[0].at[pl.ds(ebase + i * _K, _K)], src_b)
            pltpu.sync_copy(ei_hbm.at[1].at[pl.ds(ebase + i * _K, _K)], dst_b)
            pltpu.async_copy(tab_hbm.at[src_b], rows_b, sem).wait()
            pltpu.sync_copy(rows_b, acc.at[dst_b], add=True)

        plsc.subcore_barrier()
        pltpu.sync_copy(
            acc.at[pl.ds(s * _ROWS_PER_SUB, _ROWS_PER_SUB)],
            out_hbm.at[c].at[pl.ds(s * _ROWS_PER_SUB, _ROWS_PER_SUB)],
        )

    return k(ei, table)


def _agg_feat_split_call(ei, table2):
    eps = _EPAD // _NSUB
    nfull = eps // _K

    @functools.partial(
        pl.kernel,
        out_type=jax.ShapeDtypeStruct((_NCORE, _NPAD, _DIN), jnp.float32),
        mesh=_vec_mesh,
        scratch_types=[
            pltpu.VMEM((_K,), jnp.int32),
            pltpu.VMEM((_K,), jnp.int32),
            pltpu.VMEM((_K, _DIN), jnp.float32),
            pltpu.VMEM((_ZROWS, _DIN), jnp.float32),
            pltpu.VMEM_SHARED((_NPAD, _DIN), jnp.float32),
            pltpu.SemaphoreType.DMA,
        ],
    )
    def k(ei_hbm, tab_hbm, out_hbm, src_b, dst_b, rows_b, zb, acc, sem):
        c = lax.axis_index("c")
        s = lax.axis_index("s")
        _zero_fill_2d(zb, _ZROWS, _DIN)

        @pl.loop(0, _ROWS_PER_SUB // _ZROWS)
        def _(i):
            pltpu.sync_copy(
                zb, acc.at[pl.ds(s * _ROWS_PER_SUB + i * _ZROWS, _ZROWS)])

        plsc.subcore_barrier()

        ebase = s * eps

        @pl.loop(0, nfull)
        def _(i):
            pltpu.sync_copy(ei_hbm.at[0].at[pl.ds(ebase + i * _K, _K)], src_b)
            pltpu.sync_copy(ei_hbm.at[1].at[pl.ds(ebase + i * _K, _K)], dst_b)
            pltpu.async_copy(tab_hbm.at[c].at[src_b], rows_b, sem).wait()
            pltpu.sync_copy(rows_b, acc.at[dst_b], add=True)

        plsc.subcore_barrier()
        pltpu.sync_copy(
            acc.at[pl.ds(s * _ROWS_PER_SUB, _ROWS_PER_SUB)],
            out_hbm.at[c].at[pl.ds(s * _ROWS_PER_SUB, _ROWS_PER_SUB)],
        )

    return k(ei, table2)


def _scale_body(x_ref, dego_ref, o_ref):
    so = lax.rsqrt(jnp.maximum(dego_ref[...], 1.0))
    o_ref[...] = x_ref[...] * so


def _scale_call(x_pad, deg_o):
    return pl.pallas_call(
        _scale_body,
        grid=(_GRID,),
        in_specs=[
            pl.BlockSpec((_BM, _DIN), lambda i: (i, 0)),
            pl.BlockSpec((_BM, 1), lambda i: (i, 0)),
        ],
        out_specs=pl.BlockSpec((_BM, _DIN), lambda i: (i, 0)),
        out_shape=jax.ShapeDtypeStruct((_NPAD, _DIN), jnp.float32),
    )(x_pad, deg_o)


def _layer1_body(p_ref, degi_ref, dego_ref, w1_ref, b1_ref, o_ref):
    si = lax.rsqrt(jnp.maximum(degi_ref[...], 1.0))
    so = lax.rsqrt(jnp.maximum(dego_ref[...], 1.0))
    agg = (p_ref[0] + p_ref[1]) * si
    z = jnp.dot(agg, w1_ref[...], preferred_element_type=jnp.float32)
    h = jnp.maximum(z + b1_ref[...], 0.0) * so
    o_ref[0] = h[:, :_DIN]
    o_ref[1] = h[:, _DIN:]


def _layer1_call(parts, deg_i, deg_o, W1, b1):
    return pl.pallas_call(
        _layer1_body,
        grid=(_GRID,),
        in_specs=[
            pl.BlockSpec((2, _BM, _DIN), lambda i: (0, i, 0)),
            pl.BlockSpec((_BM, 1), lambda i: (i, 0)),
            pl.BlockSpec((_BM, 1), lambda i: (i, 0)),
            pl.BlockSpec((_DIN, _HID), lambda i: (0, 0)),
            pl.BlockSpec((1, _HID), lambda i: (0, 0)),
        ],
        out_specs=pl.BlockSpec((2, _BM, _DIN), lambda i: (0, i, 0)),
        out_shape=jax.ShapeDtypeStruct((2, _NPAD, _DIN), jnp.float32),
    )(parts, deg_i, deg_o, W1, b1)


def _layer2_body(agg_ref, degi_ref, w2a_ref, w2b_ref, b2_ref, wf_ref, bf_ref,
                 o_ref, acc_ref):
    b = pl.program_id(0)
    si = lax.rsqrt(jnp.maximum(degi_ref[...], 1.0))
    a0 = agg_ref[0] * si
    a1 = agg_ref[1] * si
    z = (jnp.dot(a0, w2a_ref[...], preferred_element_type=jnp.float32)
         + jnp.dot(a1, w2b_ref[...], preferred_element_type=jnp.float32)
         + b2_ref[...])
    h = jnp.maximum(z, 0.0)
    row = b * _BM + lax.broadcasted_iota(jnp.int32, (_BM, 1), 0)
    h = jnp.where(row < _N, h, 0.0)
    colsum = jnp.sum(h, axis=0, keepdims=True)

    @pl.when(b == 0)
    def _():
        acc_ref[...] = colsum

    @pl.when(b > 0)
    def _():
        acc_ref[...] = acc_ref[...] + colsum

    @pl.when(b == _GRID - 1)
    def _():
        hg = acc_ref[...] * (1.0 / _N)
        logit = jnp.dot(hg, wf_ref[...],
                        preferred_element_type=jnp.float32) + bf_ref[...]
        o_ref[...] = 1.0 / (1.0 + jnp.exp(-logit))


def _layer2_call(agg2, deg_i, W2a, W2b, b2, Wf, bf):
    return pl.pallas_call(
        _layer2_body,
        grid=(_GRID,),
        in_specs=[
            pl.BlockSpec((2, _BM, _DIN), lambda i: (0, i, 0)),
            pl.BlockSpec((_BM, 1), lambda i: (i, 0)),
            pl.BlockSpec((_DIN, _HID), lambda i: (0, 0)),
            pl.BlockSpec((_DIN, _HID), lambda i: (0, 0)),
            pl.BlockSpec((1, _HID), lambda i: (0, 0)),
            pl.BlockSpec((_HID, 1), lambda i: (0, 0)),
            pl.BlockSpec((1, 1), lambda i: (0, 0)),
        ],
        out_specs=pl.BlockSpec((1, 1), lambda i: (0, 0)),
        out_shape=jax.ShapeDtypeStruct((1, 1), jnp.float32),
        scratch_shapes=[pltpu.VMEM((1, _HID), jnp.float32)],
    )(agg2, deg_i, W2a, W2b, b2, Wf, bf)


def kernel(x, edge_index, W1, b1, W2, b2, Wf, bf):
    ei = edge_index.astype(jnp.int32)
    ei = jnp.pad(ei, ((0, 0), (0, _EPAD - _E)), constant_values=_N)
    x_pad = jnp.pad(x, ((0, _NPAD - _N), (0, 0)))

    deg = _deg_call(ei)
    deg_o = deg[0].reshape(_NPAD, 1)
    deg_i = deg[1].reshape(_NPAD, 1)

    xs = _scale_call(x_pad, deg_o)
    parts = _agg_edge_split_call(ei, xs)
    h1s = _layer1_call(parts, deg_i, deg_o, W1,
                       b1.reshape(1, _HID))
    agg2 = _agg_feat_split_call(ei, h1s)
    out = _layer2_call(agg2, deg_i, W2[:_DIN], W2[_DIN:],
                       b2.reshape(1, _HID), Wf, bf.reshape(1, 1))
    return out


# speedup vs baseline: 4.1075x; 1.0831x over previous
import functools

import jax
import jax.numpy as jnp
from jax import lax
from jax.experimental import pallas as pl
from jax.experimental.pallas import tpu as pltpu
from jax.experimental.pallas import tpu_sc as plsc

_N = 10000
_E = 320000
_DIN = 128
_HID = 256
_NPAD = 10240
_EPAD = 327680
_NSUB = 16
_NCORE = 2
_K = 128
_GRP = 16
_ECHUNKS = _EPAD // _K
_ROWS_PER_SUB = _NPAD // _NSUB
_ZROWS = 64
_BM = 1024
_GRID = _NPAD // _BM

_vec_mesh = plsc.VectorSubcoreMesh(core_axis_name="c", subcore_axis_name="s")


def _zero_fill_1d(buf, n):
    @pl.loop(0, n // 16)
    def _(i):
        buf[pl.ds(i * 16, 16)] = jnp.zeros((16,), jnp.float32)


def _zero_fill_2d(buf, rows, cols):
    @pl.loop(0, rows)
    def _(r):
        @pl.loop(0, cols // 16)
        def _(j):
            buf[r, pl.ds(j * 16, 16)] = jnp.zeros((16,), jnp.float32)


def _deg_call(src2, dst2):
    chunks = _ECHUNKS // _NSUB

    @functools.partial(
        pl.kernel,
        out_type=jax.ShapeDtypeStruct((_NCORE, _NPAD), jnp.float32),
        mesh=_vec_mesh,
        scratch_types=[
            pltpu.VMEM((chunks, _K), jnp.int32),
            pltpu.VMEM((_K,), jnp.float32),
            pltpu.VMEM((_ROWS_PER_SUB,), jnp.float32),
            pltpu.VMEM_SHARED((_NPAD,), jnp.float32),
            pltpu.SemaphoreType.DMA,
            pltpu.SemaphoreType.DMA,
        ],
    )
    def k(src_hbm, dst_hbm, out_hbm, idx_b, ones_b, zb, acc, isem, ssem):
        c = lax.axis_index("c")
        s = lax.axis_index("s")
        _zero_fill_1d(zb, _ROWS_PER_SUB)

        @pl.loop(0, _K // 16)
        def _(i):
            ones_b[pl.ds(i * 16, 16)] = jnp.ones((16,), jnp.float32)

        cbase = s * chunks

        @pl.when(c == 0)
        def _():
            pltpu.async_copy(src_hbm.at[pl.ds(cbase, chunks)], idx_b, isem)

        @pl.when(c == 1)
        def _():
            pltpu.async_copy(dst_hbm.at[pl.ds(cbase, chunks)], idx_b, isem)

        pltpu.sync_copy(zb, acc.at[pl.ds(s * _ROWS_PER_SUB, _ROWS_PER_SUB)])
        pltpu.make_async_copy(src_hbm.at[pl.ds(cbase, chunks)], idx_b,
                              isem).wait()
        plsc.subcore_barrier()

        @pl.loop(0, 8)
        def _(t):
            pltpu.async_copy(ones_b, acc.at[idx_b.at[t]], ssem, add=True)

        @pl.loop(8, chunks)
        def _(t):
            pltpu.make_async_copy(ones_b, acc.at[idx_b.at[t - 8]],
                                  ssem).wait()
            pltpu.async_copy(ones_b, acc.at[idx_b.at[t]], ssem, add=True)

        @pl.loop(chunks - 8, chunks)
        def _(t):
            pltpu.make_async_copy(ones_b, acc.at[idx_b.at[t]], ssem).wait()

        plsc.subcore_barrier()
        pltpu.sync_copy(
            acc.at[pl.ds(s * _ROWS_PER_SUB, _ROWS_PER_SUB)],
            out_hbm.at[c].at[pl.ds(s * _ROWS_PER_SUB, _ROWS_PER_SUB)],
        )

    return k(src2, dst2)


def _make_agg(edge_split):
    chunks = _ECHUNKS // (_NSUB * (_NCORE if edge_split else 1))
    ngroups = chunks // _GRP

    @functools.partial(
        pl.kernel,
        out_type=jax.ShapeDtypeStruct((_NCORE, _NPAD, _DIN), jnp.float32),
        mesh=_vec_mesh,
        scratch_types=[
            pltpu.VMEM((_GRP, _K), jnp.int32),
            pltpu.VMEM((_GRP, _K), jnp.int32),
            pltpu.VMEM((_K, _DIN), jnp.float32),
            pltpu.VMEM((_K, _DIN), jnp.float32),
            pltpu.VMEM((_ZROWS, _DIN), jnp.float32),
            pltpu.VMEM_SHARED((_NPAD, _DIN), jnp.float32),
            pltpu.SemaphoreType.DMA,
            pltpu.SemaphoreType.DMA,
            pltpu.SemaphoreType.DMA,
            pltpu.SemaphoreType.DMA,
            pltpu.SemaphoreType.DMA,
        ],
    )
    def k(src_hbm, dst_hbm, tab_hbm, out_hbm, src_b, dst_b, rows0, rows1,
          zb, acc, isem0, isem1, gsem0, gsem1, ssem0):
        c = lax.axis_index("c")
        s = lax.axis_index("s")
        rows = (rows0, rows1)
        gsem = (gsem0, gsem1)
        ssem = (isem1, ssem0)

        if edge_split:
            cbase = (c * _NSUB + s) * chunks
            tab = tab_hbm
        else:
            cbase = s * chunks
            tab = tab_hbm.at[c]

        _zero_fill_2d(zb, _ZROWS, _DIN)

        @pl.loop(0, _ROWS_PER_SUB // _ZROWS)
        def _(i):
            pltpu.sync_copy(
                zb, acc.at[pl.ds(s * _ROWS_PER_SUB + i * _ZROWS, _ZROWS)])

        plsc.subcore_barrier()

        def g_start(t, p):
            pltpu.async_copy(tab.at[src_b.at[t]], rows[p], gsem[p])

        def g_wait(t, p):
            pltpu.make_async_copy(tab.at[src_b.at[t]], rows[p],
                                  gsem[p]).wait()

        def s_start(t, p):
            pltpu.async_copy(rows[p], acc.at[dst_b.at[t]], ssem[p], add=True)

        def s_wait(t, p):
            pltpu.make_async_copy(rows[p], acc.at[dst_b.at[t]],
                                  ssem[p]).wait()

        @pl.loop(0, ngroups)
        def _(g):
            gb = cbase + g * _GRP
            pltpu.async_copy(src_hbm.at[pl.ds(gb, _GRP)], src_b, isem0)
            pltpu.async_copy(dst_hbm.at[pl.ds(gb, _GRP)], dst_b, gsem0)
            pltpu.make_async_copy(src_hbm.at[pl.ds(gb, _GRP)], src_b,
                                  isem0).wait()
            pltpu.make_async_copy(dst_hbm.at[pl.ds(gb, _GRP)], dst_b,
                                  gsem0).wait()

            g_start(0, 0)
            g_start(1, 1)
            g_wait(0, 0)
            s_start(0, 0)

            @pl.loop(1, _GRP // 2)
            def _(j):
                for b in range(2):
                    t = 2 * j + b
                    p = b
                    q = 1 - b
                    s_wait(t - 2, p)
                    g_start(t, p)
                    g_wait(t - 1, q)
                    s_start(t - 1, q)

            s_wait(_GRP - 2, 0)
            g_wait(_GRP - 1, 1)
            s_start(_GRP - 1, 1)
            s_wait(_GRP - 1, 1)

        plsc.subcore_barrier()
        pltpu.sync_copy(
            acc.at[pl.ds(s * _ROWS_PER_SUB, _ROWS_PER_SUB)],
            out_hbm.at[c].at[pl.ds(s * _ROWS_PER_SUB, _ROWS_PER_SUB)],
        )

    return k


_agg_edge_split_call = _make_agg(True)
_agg_feat_split_call = _make_agg(False)


def _scale_body(x_ref, dego_ref, o_ref):
    so = lax.rsqrt(jnp.maximum(dego_ref[...], 1.0))
    o_ref[...] = x_ref[...] * so


def _scale_call(x_pad, deg_o):
    return pl.pallas_call(
        _scale_body,
        grid=(_GRID,),
        in_specs=[
            pl.BlockSpec((_BM, _DIN), lambda i: (i, 0)),
            pl.BlockSpec((_BM, 1), lambda i: (i, 0)),
        ],
        out_specs=pl.BlockSpec((_BM, _DIN), lambda i: (i, 0)),
        out_shape=jax.ShapeDtypeStruct((_NPAD, _DIN), jnp.float32),
    )(x_pad, deg_o)


def _layer1_body(p_ref, degi_ref, dego_ref, w1_ref, b1_ref, o_ref):
    si = lax.rsqrt(jnp.maximum(degi_ref[...], 1.0))
    so = lax.rsqrt(jnp.maximum(dego_ref[...], 1.0))
    agg = (p_ref[0] + p_ref[1]) * si
    z = jnp.dot(agg, w1_ref[...], preferred_element_type=jnp.float32)
    h = jnp.maximum(z + b1_ref[...], 0.0) * so
    o_ref[0] = h[:, :_DIN]
    o_ref[1] = h[:, _DIN:]


def _layer1_call(parts, deg_i, deg_o, W1, b1):
    return pl.pallas_call(
        _layer1_body,
        grid=(_GRID,),
        in_specs=[
            pl.BlockSpec((2, _BM, _DIN), lambda i: (0, i, 0)),
            pl.BlockSpec((_BM, 1), lambda i: (i, 0)),
            pl.BlockSpec((_BM, 1), lambda i: (i, 0)),
            pl.BlockSpec((_DIN, _HID), lambda i: (0, 0)),
            pl.BlockSpec((1, _HID), lambda i: (0, 0)),
        ],
        out_specs=pl.BlockSpec((2, _BM, _DIN), lambda i: (0, i, 0)),
        out_shape=jax.ShapeDtypeStruct((2, _NPAD, _DIN), jnp.float32),
    )(parts, deg_i, deg_o, W1, b1)


def _layer2_body(agg_ref, degi_ref, w2a_ref, w2b_ref, b2_ref, wf_ref, bf_ref,
                 o_ref, acc_ref):
    b = pl.program_id(0)
    si = lax.rsqrt(jnp.maximum(degi_ref[...], 1.0))
    a0 = agg_ref[0] * si
    a1 = agg_ref[1] * si
    z = (jnp.dot(a0, w2a_ref[...], preferred_element_type=jnp.float32)
         + jnp.dot(a1, w2b_ref[...], preferred_element_type=jnp.float32)
         + b2_ref[...])
    h = jnp.maximum(z, 0.0)
    row = b * _BM + lax.broadcasted_iota(jnp.int32, (_BM, 1), 0)
    h = jnp.where(row < _N, h, 0.0)
    colsum = jnp.sum(h, axis=0, keepdims=True)

    @pl.when(b == 0)
    def _():
        acc_ref[...] = colsum

    @pl.when(b > 0)
    def _():
        acc_ref[...] = acc_ref[...] + colsum

    @pl.when(b == _GRID - 1)
    def _():
        hg = acc_ref[...] * (1.0 / _N)
        logit = jnp.dot(hg, wf_ref[...],
                        preferred_element_type=jnp.float32) + bf_ref[...]
        o_ref[...] = 1.0 / (1.0 + jnp.exp(-logit))


def _layer2_call(agg2, deg_i, W2a, W2b, b2, Wf, bf):
    return pl.pallas_call(
        _layer2_body,
        grid=(_GRID,),
        in_specs=[
            pl.BlockSpec((2, _BM, _DIN), lambda i: (0, i, 0)),
            pl.BlockSpec((_BM, 1), lambda i: (i, 0)),
            pl.BlockSpec((_DIN, _HID), lambda i: (0, 0)),
            pl.BlockSpec((_DIN, _HID), lambda i: (0, 0)),
            pl.BlockSpec((1, _HID), lambda i: (0, 0)),
            pl.BlockSpec((_HID, 1), lambda i: (0, 0)),
            pl.BlockSpec((1, 1), lambda i: (0, 0)),
        ],
        out_specs=pl.BlockSpec((1, 1), lambda i: (0, 0)),
        out_shape=jax.ShapeDtypeStruct((1, 1), jnp.float32),
        scratch_shapes=[pltpu.VMEM((1, _HID), jnp.float32)],
    )(agg2, deg_i, W2a, W2b, b2, Wf, bf)


def kernel(x, edge_index, W1, b1, W2, b2, Wf, bf):
    ei = edge_index.astype(jnp.int32)
    ei = jnp.pad(ei, ((0, 0), (0, _EPAD - _E)), constant_values=_N)
    src2 = ei[0].reshape(_ECHUNKS, _K)
    dst2 = ei[1].reshape(_ECHUNKS, _K)
    x_pad = jnp.pad(x, ((0, _NPAD - _N), (0, 0)))

    deg = _deg_call(src2, dst2)
    deg_o = deg[0].reshape(_NPAD, 1)
    deg_i = deg[1].reshape(_NPAD, 1)

    xs = _scale_call(x_pad, deg_o)
    parts = _agg_edge_split_call(src2, dst2, xs)
    h1s = _layer1_call(parts, deg_i, deg_o, W1,
                       b1.reshape(1, _HID))
    agg2 = _agg_feat_split_call(src2, dst2, h1s)
    out = _layer2_call(agg2, deg_i, W2[:_DIN], W2[_DIN:],
                       b2.reshape(1, _HID), Wf, bf.reshape(1, 1))
    return out


# speedup vs baseline: 11.2745x; 2.7448x over previous
import functools

import jax
import jax.numpy as jnp
from jax import lax
from jax.experimental import pallas as pl
from jax.experimental.pallas import tpu as pltpu
from jax.experimental.pallas import tpu_sc as plsc

_N = 10000
_E = 320000
_DIN = 128
_HID = 256
_NPAD = 10240
_EPAD = 327680
_NSUB = 16
_NCORE = 2
_K = 128
_GRP = 16
_ECHUNKS = _EPAD // _K
_ROWS_PER_SUB = _NPAD // _NSUB
_ZROWS = 64
_BM = 1024
_GRID = _NPAD // _BM

_vec_mesh = plsc.VectorSubcoreMesh(core_axis_name="c", subcore_axis_name="s")


def _zero_fill_1d(buf, n):
    @pl.loop(0, n // 16)
    def _(i):
        buf[pl.ds(i * 16, 16)] = jnp.zeros((16,), jnp.float32)


def _zero_fill_2d(buf, rows, cols):
    @pl.loop(0, rows)
    def _(r):
        @pl.loop(0, cols // 16)
        def _(j):
            buf[r, pl.ds(j * 16, 16)] = jnp.zeros((16,), jnp.float32)


def _deg_call(src2, dst2):
    chunks = _ECHUNKS // _NSUB

    @functools.partial(
        pl.kernel,
        out_type=jax.ShapeDtypeStruct((_NCORE, _NPAD), jnp.float32),
        mesh=_vec_mesh,
        scratch_types=[
            pltpu.VMEM((chunks, _K), jnp.int32),
            pltpu.VMEM((_K,), jnp.float32),
            pltpu.VMEM((_ROWS_PER_SUB,), jnp.float32),
            pltpu.VMEM_SHARED((_NPAD,), jnp.float32),
            pltpu.SemaphoreType.DMA,
            pltpu.SemaphoreType.DMA,
        ],
    )
    def k(src_hbm, dst_hbm, out_hbm, idx_b, ones_b, zb, acc, isem, ssem):
        c = lax.axis_index("c")
        s = lax.axis_index("s")
        _zero_fill_1d(zb, _ROWS_PER_SUB)

        @pl.loop(0, _K // 16)
        def _(i):
            ones_b[pl.ds(i * 16, 16)] = jnp.ones((16,), jnp.float32)

        cbase = s * chunks

        @pl.when(c == 0)
        def _():
            pltpu.async_copy(src_hbm.at[pl.ds(cbase, chunks)], idx_b, isem)

        @pl.when(c == 1)
        def _():
            pltpu.async_copy(dst_hbm.at[pl.ds(cbase, chunks)], idx_b, isem)

        pltpu.sync_copy(zb, acc.at[pl.ds(s * _ROWS_PER_SUB, _ROWS_PER_SUB)])
        pltpu.make_async_copy(src_hbm.at[pl.ds(cbase, chunks)], idx_b,
                              isem).wait()
        plsc.subcore_barrier()

        @pl.loop(0, 8)
        def _(t):
            pltpu.async_copy(ones_b, acc.at[idx_b.at[t]], ssem, add=True)

        @pl.loop(8, chunks)
        def _(t):
            pltpu.make_async_copy(ones_b, acc.at[idx_b.at[t - 8]],
                                  ssem).wait()
            pltpu.async_copy(ones_b, acc.at[idx_b.at[t]], ssem, add=True)

        @pl.loop(chunks - 8, chunks)
        def _(t):
            pltpu.make_async_copy(ones_b, acc.at[idx_b.at[t]], ssem).wait()

        plsc.subcore_barrier()
        pltpu.sync_copy(
            acc.at[pl.ds(s * _ROWS_PER_SUB, _ROWS_PER_SUB)],
            out_hbm.at[c].at[pl.ds(s * _ROWS_PER_SUB, _ROWS_PER_SUB)],
        )

    return k(src2, dst2)


def _make_agg(edge_split):
    chunks = _ECHUNKS // (_NSUB * (_NCORE if edge_split else 1))
    ngroups = chunks // _GRP

    @functools.partial(
        pl.kernel,
        out_type=jax.ShapeDtypeStruct((_NCORE, _NPAD, _DIN), jnp.float32),
        mesh=_vec_mesh,
        scratch_types=[
            pltpu.VMEM((_GRP, _K), jnp.int32),
            pltpu.VMEM((_GRP, _K), jnp.int32),
            pltpu.VMEM((_K, _DIN), jnp.float32),
            pltpu.VMEM((_K, _DIN), jnp.float32),
            pltpu.VMEM((_ZROWS, _DIN), jnp.float32),
            pltpu.VMEM_SHARED((_NPAD, _DIN), jnp.float32),
            pltpu.SemaphoreType.DMA,
            pltpu.SemaphoreType.DMA,
            pltpu.SemaphoreType.DMA,
            pltpu.SemaphoreType.DMA,
            pltpu.SemaphoreType.DMA,
        ],
    )
    def k(src_hbm, dst_hbm, tab_hbm, out_hbm, src_b, dst_b, rows0, rows1,
          zb, acc, isem0, isem1, gsem0, gsem1, ssem0):
        c = lax.axis_index("c")
        s = lax.axis_index("s")
        rows = (rows0, rows1)
        gsem = (gsem0, gsem1)
        ssem = (isem1, ssem0)

        if edge_split:
            cbase = (c * _NSUB + s) * chunks
            tab = tab_hbm
        else:
            cbase = s * chunks
            tab = tab_hbm.at[c]

        _zero_fill_2d(zb, _ZROWS, _DIN)

        @pl.loop(0, _ROWS_PER_SUB // _ZROWS)
        def _(i):
            pltpu.sync_copy(
                zb, acc.at[pl.ds(s * _ROWS_PER_SUB + i * _ZROWS, _ZROWS)])

        plsc.subcore_barrier()

        def g_start(t, p):
            pltpu.async_copy(tab.at[src_b.at[t]], rows[p], gsem[p])

        def g_wait(t, p):
            pltpu.make_async_copy(tab.at[src_b.at[t]], rows[p],
                                  gsem[p]).wait()

        def s_start(t, p):
            pltpu.async_copy(rows[p], acc.at[dst_b.at[t]], ssem[p], add=True)

        def s_wait(t, p):
            pltpu.make_async_copy(rows[p], acc.at[dst_b.at[t]],
                                  ssem[p]).wait()

        @pl.loop(0, ngroups)
        def _(g):
            gb = cbase + g * _GRP
            pltpu.async_copy(src_hbm.at[pl.ds(gb, _GRP)], src_b, isem0)
            pltpu.async_copy(dst_hbm.at[pl.ds(gb, _GRP)], dst_b, gsem0)
            pltpu.make_async_copy(src_hbm.at[pl.ds(gb, _GRP)], src_b,
                                  isem0).wait()
            pltpu.make_async_copy(dst_hbm.at[pl.ds(gb, _GRP)], dst_b,
                                  gsem0).wait()

            g_start(0, 0)
            g_start(1, 1)
            g_wait(0, 0)
            s_start(0, 0)

            @pl.loop(1, _GRP // 2)
            def _(j):
                for b in range(2):
                    t = 2 * j + b
                    p = b
                    q = 1 - b
                    s_wait(t - 2, p)
                    g_start(t, p)
                    g_wait(t - 1, q)
                    s_start(t - 1, q)

            s_wait(_GRP - 2, 0)
            g_wait(_GRP - 1, 1)
            s_start(_GRP - 1, 1)
            s_wait(_GRP - 1, 1)

        plsc.subcore_barrier()
        pltpu.sync_copy(
            acc.at[pl.ds(s * _ROWS_PER_SUB, _ROWS_PER_SUB)],
            out_hbm.at[c].at[pl.ds(s * _ROWS_PER_SUB, _ROWS_PER_SUB)],
        )

    return k


_agg_edge_split_call = _make_agg(True)
_agg_feat_split_call = _make_agg(False)


def _scale_body(x_ref, dego_ref, o_ref):
    so = lax.rsqrt(jnp.maximum(dego_ref[...], 1.0))
    o_ref[...] = x_ref[...] * so


def _scale_call(x_pad, deg_o):
    return pl.pallas_call(
        _scale_body,
        grid=(_GRID,),
        in_specs=[
            pl.BlockSpec((_BM, _DIN), lambda i: (i, 0)),
            pl.BlockSpec((_BM, 1), lambda i: (i, 0)),
        ],
        out_specs=pl.BlockSpec((_BM, _DIN), lambda i: (i, 0)),
        out_shape=jax.ShapeDtypeStruct((_NPAD, _DIN), jnp.float32),
    )(x_pad, deg_o)


def _layer1_body(p_ref, degi_ref, dego_ref, w1_ref, b1_ref, o_ref):
    si = lax.rsqrt(jnp.maximum(degi_ref[...], 1.0))
    so = lax.rsqrt(jnp.maximum(dego_ref[...], 1.0))
    agg = (p_ref[0] + p_ref[1]) * si
    z = jnp.dot(agg, w1_ref[...], preferred_element_type=jnp.float32)
    h = jnp.maximum(z + b1_ref[...], 0.0) * so
    o_ref[0] = h[:, :_DIN]
    o_ref[1] = h[:, _DIN:]


def _layer1_call(parts, deg_i, deg_o, W1, b1):
    return pl.pallas_call(
        _layer1_body,
        grid=(_GRID,),
        in_specs=[
            pl.BlockSpec((2, _BM, _DIN), lambda i: (0, i, 0)),
            pl.BlockSpec((_BM, 1), lambda i: (i, 0)),
            pl.BlockSpec((_BM, 1), lambda i: (i, 0)),
            pl.BlockSpec((_DIN, _HID), lambda i: (0, 0)),
            pl.BlockSpec((1, _HID), lambda i: (0, 0)),
        ],
        out_specs=pl.BlockSpec((2, _BM, _DIN), lambda i: (0, i, 0)),
        out_shape=jax.ShapeDtypeStruct((2, _NPAD, _DIN), jnp.float32),
    )(parts, deg_i, deg_o, W1, b1)


def _layer2_body(agg_ref, degi_ref, w2a_ref, w2b_ref, b2_ref, wf_ref, bf_ref,
                 o_ref, acc_ref):
    b = pl.program_id(0)
    si = lax.rsqrt(jnp.maximum(degi_ref[...], 1.0))
    a0 = agg_ref[0] * si
    a1 = agg_ref[1] * si
    z = (jnp.dot(a0, w2a_ref[...], preferred_element_type=jnp.float32)
         + jnp.dot(a1, w2b_ref[...], preferred_element_type=jnp.float32)
         + b2_ref[...])
    h = jnp.maximum(z, 0.0)
    row = b * _BM + lax.broadcasted_iota(jnp.int32, (_BM, 1), 0)
    h = jnp.where(row < _N, h, 0.0)
    colsum = jnp.sum(h, axis=0, keepdims=True)

    @pl.when(b == 0)
    def _():
        acc_ref[...] = colsum

    @pl.when(b > 0)
    def _():
        acc_ref[...] = acc_ref[...] + colsum

    @pl.when(b == _GRID - 1)
    def _():
        hg = acc_ref[...] * (1.0 / _N)
        logit = jnp.dot(hg, wf_ref[...],
                        preferred_element_type=jnp.float32) + bf_ref[...]
        o_ref[...] = 1.0 / (1.0 + jnp.exp(-logit))


def _layer2_call(agg2, deg_i, W2a, W2b, b2, Wf, bf):
    return pl.pallas_call(
        _layer2_body,
        grid=(_GRID,),
        in_specs=[
            pl.BlockSpec((2, _BM, _DIN), lambda i: (0, i, 0)),
            pl.BlockSpec((_BM, 1), lambda i: (i, 0)),
            pl.BlockSpec((_DIN, _HID), lambda i: (0, 0)),
            pl.BlockSpec((_DIN, _HID), lambda i: (0, 0)),
            pl.BlockSpec((1, _HID), lambda i: (0, 0)),
            pl.BlockSpec((_HID, 1), lambda i: (0, 0)),
            pl.BlockSpec((1, 1), lambda i: (0, 0)),
        ],
        out_specs=pl.BlockSpec((1, 1), lambda i: (0, 0)),
        out_shape=jax.ShapeDtypeStruct((1, 1), jnp.float32),
        scratch_shapes=[pltpu.VMEM((1, _HID), jnp.float32)],
    )(agg2, deg_i, W2a, W2b, b2, Wf, bf)


def kernel(x, edge_index, W1, b1, W2, b2, Wf, bf):
    ei = edge_index.astype(jnp.int32)
    pad_fill = _N + jnp.arange(_EPAD - _E, dtype=jnp.int32) % (_NPAD - _N)
    ei = jnp.concatenate(
        [ei, jnp.stack([pad_fill, pad_fill])], axis=1)
    src2 = ei[0].reshape(_ECHUNKS, _K)
    dst2 = ei[1].reshape(_ECHUNKS, _K)
    x_pad = jnp.pad(x, ((0, _NPAD - _N), (0, 0)))

    deg = _deg_call(src2, dst2)
    deg_o = deg[0].reshape(_NPAD, 1)
    deg_i = deg[1].reshape(_NPAD, 1)

    xs = _scale_call(x_pad, deg_o)
    parts = _agg_edge_split_call(src2, dst2, xs)
    h1s = _layer1_call(parts, deg_i, deg_o, W1,
                       b1.reshape(1, _HID))
    agg2 = _agg_feat_split_call(src2, dst2, h1s)
    out = _layer2_call(agg2, deg_i, W2[:_DIN], W2[_DIN:],
                       b2.reshape(1, _HID), Wf, bf.reshape(1, 1))
    return out


# speedup vs baseline: 11.6287x; 1.0314x over previous
import functools

import jax
import jax.numpy as jnp
from jax import lax
from jax.experimental import pallas as pl
from jax.experimental.pallas import tpu as pltpu
from jax.experimental.pallas import tpu_sc as plsc

_N = 10000
_E = 320000
_DIN = 128
_HID = 256
_NPAD = 10240
_EPAD = 327680
_NSUB = 16
_NCORE = 2
_K = 128
_GRP = 16
_ECHUNKS = _EPAD // _K
_ROWS_PER_SUB = _NPAD // _NSUB
_ZROWS = 32
_BM = 1024
_GRID = _NPAD // _BM

_vec_mesh = plsc.VectorSubcoreMesh(core_axis_name="c", subcore_axis_name="s")


def _zero_fill_1d(buf, n):
    @pl.loop(0, n // 16)
    def _(i):
        buf[pl.ds(i * 16, 16)] = jnp.zeros((16,), jnp.float32)


def _zero_fill_2d(buf, rows, cols):
    @pl.loop(0, rows)
    def _(r):
        @pl.loop(0, cols // 16)
        def _(j):
            buf[r, pl.ds(j * 16, 16)] = jnp.zeros((16,), jnp.float32)


def _deg_call(src2, dst2):
    chunks = _ECHUNKS // _NSUB

    @functools.partial(
        pl.kernel,
        out_type=jax.ShapeDtypeStruct((_NCORE, _NPAD), jnp.float32),
        mesh=_vec_mesh,
        scratch_types=[
            pltpu.VMEM((chunks, _K), jnp.int32),
            pltpu.VMEM((_K,), jnp.float32),
            pltpu.VMEM((_ROWS_PER_SUB,), jnp.float32),
            pltpu.VMEM_SHARED((_NPAD,), jnp.float32),
            pltpu.SemaphoreType.DMA,
            pltpu.SemaphoreType.DMA,
        ],
    )
    def k(src_hbm, dst_hbm, out_hbm, idx_b, ones_b, zb, acc, isem, ssem):
        c = lax.axis_index("c")
        s = lax.axis_index("s")
        _zero_fill_1d(zb, _ROWS_PER_SUB)

        @pl.loop(0, _K // 16)
        def _(i):
            ones_b[pl.ds(i * 16, 16)] = jnp.ones((16,), jnp.float32)

        cbase = s * chunks

        @pl.when(c == 0)
        def _():
            pltpu.async_copy(src_hbm.at[pl.ds(cbase, chunks)], idx_b, isem)

        @pl.when(c == 1)
        def _():
            pltpu.async_copy(dst_hbm.at[pl.ds(cbase, chunks)], idx_b, isem)

        pltpu.sync_copy(zb, acc.at[pl.ds(s * _ROWS_PER_SUB, _ROWS_PER_SUB)])
        pltpu.make_async_copy(src_hbm.at[pl.ds(cbase, chunks)], idx_b,
                              isem).wait()
        plsc.subcore_barrier()

        @pl.loop(0, 8)
        def _(t):
            pltpu.async_copy(ones_b, acc.at[idx_b.at[t]], ssem, add=True)

        @pl.loop(8, chunks)
        def _(t):
            pltpu.make_async_copy(ones_b, acc.at[idx_b.at[t - 8]],
                                  ssem).wait()
            pltpu.async_copy(ones_b, acc.at[idx_b.at[t]], ssem, add=True)

        @pl.loop(chunks - 8, chunks)
        def _(t):
            pltpu.make_async_copy(ones_b, acc.at[idx_b.at[t]], ssem).wait()

        plsc.subcore_barrier()
        pltpu.sync_copy(
            acc.at[pl.ds(s * _ROWS_PER_SUB, _ROWS_PER_SUB)],
            out_hbm.at[c].at[pl.ds(s * _ROWS_PER_SUB, _ROWS_PER_SUB)],
        )

    return k(src2, dst2)


def _make_agg(edge_split):
    chunks = _ECHUNKS // (_NSUB * (_NCORE if edge_split else 1))
    grp = _GRP if edge_split else 2 * _GRP
    ngroups = chunks // grp

    @functools.partial(
        pl.kernel,
        out_type=jax.ShapeDtypeStruct((_NCORE, _NPAD, _DIN), jnp.float32),
        mesh=_vec_mesh,
        scratch_types=[
            pltpu.VMEM((grp, _K), jnp.int32),
            pltpu.VMEM((grp, _K), jnp.int32),
            pltpu.VMEM((_K, _DIN), jnp.float32),
            pltpu.VMEM((_K, _DIN), jnp.float32),
            pltpu.VMEM((_ZROWS, _DIN), jnp.float32),
            pltpu.VMEM_SHARED((_NPAD, _DIN), jnp.float32),
            pltpu.SemaphoreType.DMA,
            pltpu.SemaphoreType.DMA,
            pltpu.SemaphoreType.DMA,
            pltpu.SemaphoreType.DMA,
            pltpu.SemaphoreType.DMA,
        ],
    )
    def k(src_hbm, dst_hbm, tab_hbm, out_hbm, src_b, dst_b, rows0, rows1,
          zb, acc, isem0, isem1, gsem0, gsem1, ssem0):
        c = lax.axis_index("c")
        s = lax.axis_index("s")
        rows = (rows0, rows1)
        gsem = (gsem0, gsem1)
        ssem = (isem1, ssem0)

        if edge_split:
            cbase = (c * _NSUB + s) * chunks
            tab = tab_hbm
        else:
            cbase = s * chunks
            tab = tab_hbm.at[c]

        _zero_fill_2d(zb, _ZROWS, _DIN)

        @pl.loop(0, _ROWS_PER_SUB // _ZROWS)
        def _(i):
            pltpu.sync_copy(
                zb, acc.at[pl.ds(s * _ROWS_PER_SUB + i * _ZROWS, _ZROWS)])

        plsc.subcore_barrier()

        def g_start(t, p):
            pltpu.async_copy(tab.at[src_b.at[t]], rows[p], gsem[p])

        def g_wait(t, p):
            pltpu.make_async_copy(tab.at[src_b.at[t]], rows[p],
                                  gsem[p]).wait()

        def s_start(t, p):
            pltpu.async_copy(rows[p], acc.at[dst_b.at[t]], ssem[p], add=True)

        def s_wait(t, p):
            pltpu.make_async_copy(rows[p], acc.at[dst_b.at[t]],
                                  ssem[p]).wait()

        @pl.loop(0, ngroups)
        def _(g):
            gb = cbase + g * grp
            pltpu.async_copy(src_hbm.at[pl.ds(gb, grp)], src_b, isem0)
            pltpu.async_copy(dst_hbm.at[pl.ds(gb, grp)], dst_b, gsem0)
            pltpu.make_async_copy(src_hbm.at[pl.ds(gb, grp)], src_b,
                                  isem0).wait()
            pltpu.make_async_copy(dst_hbm.at[pl.ds(gb, grp)], dst_b,
                                  gsem0).wait()

            g_start(0, 0)
            g_start(1, 1)
            g_wait(0, 0)
            s_start(0, 0)

            @pl.loop(1, grp // 2)
            def _(j):
                for b in range(2):
                    t = 2 * j + b
                    p = b
                    q = 1 - b
                    s_wait(t - 2, p)
                    g_start(t, p)
                    g_wait(t - 1, q)
                    s_start(t - 1, q)

            s_wait(grp - 2, 0)
            g_wait(grp - 1, 1)
            s_start(grp - 1, 1)
            s_wait(grp - 1, 1)

        plsc.subcore_barrier()
        pltpu.sync_copy(
            acc.at[pl.ds(s * _ROWS_PER_SUB, _ROWS_PER_SUB)],
            out_hbm.at[c].at[pl.ds(s * _ROWS_PER_SUB, _ROWS_PER_SUB)],
        )

    return k


_agg_edge_split_call = _make_agg(True)
_agg_feat_split_call = _make_agg(False)


def _scale_body(x_ref, dego_ref, o_ref):
    so = lax.rsqrt(jnp.maximum(dego_ref[...], 1.0))
    o_ref[...] = x_ref[...] * so


def _scale_call(x_pad, deg_o):
    return pl.pallas_call(
        _scale_body,
        grid=(_GRID,),
        in_specs=[
            pl.BlockSpec((_BM, _DIN), lambda i: (i, 0)),
            pl.BlockSpec((_BM, 1), lambda i: (i, 0)),
        ],
        out_specs=pl.BlockSpec((_BM, _DIN), lambda i: (i, 0)),
        out_shape=jax.ShapeDtypeStruct((_NPAD, _DIN), jnp.float32),
    )(x_pad, deg_o)


def _layer1_body(p_ref, degi_ref, dego_ref, w1_ref, b1_ref, o_ref):
    si = lax.rsqrt(jnp.maximum(degi_ref[...], 1.0))
    so = lax.rsqrt(jnp.maximum(dego_ref[...], 1.0))
    agg = (p_ref[0] + p_ref[1]) * si
    z = jnp.dot(agg, w1_ref[...], preferred_element_type=jnp.float32)
    h = jnp.maximum(z + b1_ref[...], 0.0) * so
    o_ref[0] = h[:, :_DIN]
    o_ref[1] = h[:, _DIN:]


def _layer1_call(parts, deg_i, deg_o, W1, b1):
    return pl.pallas_call(
        _layer1_body,
        grid=(_GRID,),
        in_specs=[
            pl.BlockSpec((2, _BM, _DIN), lambda i: (0, i, 0)),
            pl.BlockSpec((_BM, 1), lambda i: (i, 0)),
            pl.BlockSpec((_BM, 1), lambda i: (i, 0)),
            pl.BlockSpec((_DIN, _HID), lambda i: (0, 0)),
            pl.BlockSpec((1, _HID), lambda i: (0, 0)),
        ],
        out_specs=pl.BlockSpec((2, _BM, _DIN), lambda i: (0, i, 0)),
        out_shape=jax.ShapeDtypeStruct((2, _NPAD, _DIN), jnp.float32),
    )(parts, deg_i, deg_o, W1, b1)


def _layer2_body(agg_ref, degi_ref, w2a_ref, w2b_ref, b2_ref, wf_ref, bf_ref,
                 o_ref, acc_ref):
    b = pl.program_id(0)
    si = lax.rsqrt(jnp.maximum(degi_ref[...], 1.0))
    a0 = agg_ref[0] * si
    a1 = agg_ref[1] * si
    z = (jnp.dot(a0, w2a_ref[...], preferred_element_type=jnp.float32)
         + jnp.dot(a1, w2b_ref[...], preferred_element_type=jnp.float32)
         + b2_ref[...])
    h = jnp.maximum(z, 0.0)
    row = b * _BM + lax.broadcasted_iota(jnp.int32, (_BM, 1), 0)
    h = jnp.where(row < _N, h, 0.0)
    colsum = jnp.sum(h, axis=0, keepdims=True)

    @pl.when(b == 0)
    def _():
        acc_ref[...] = colsum

    @pl.when(b > 0)
    def _():
        acc_ref[...] = acc_ref[...] + colsum

    @pl.when(b == _GRID - 1)
    def _():
        hg = acc_ref[...] * (1.0 / _N)
        logit = jnp.dot(hg, wf_ref[...],
                        preferred_element_type=jnp.float32) + bf_ref[...]
        o_ref[...] = 1.0 / (1.0 + jnp.exp(-logit))


def _layer2_call(agg2, deg_i, W2a, W2b, b2, Wf, bf):
    return pl.pallas_call(
        _layer2_body,
        grid=(_GRID,),
        in_specs=[
            pl.BlockSpec((2, _BM, _DIN), lambda i: (0, i, 0)),
            pl.BlockSpec((_BM, 1), lambda i: (i, 0)),
            pl.BlockSpec((_DIN, _HID), lambda i: (0, 0)),
            pl.BlockSpec((_DIN, _HID), lambda i: (0, 0)),
            pl.BlockSpec((1, _HID), lambda i: (0, 0)),
            pl.BlockSpec((_HID, 1), lambda i: (0, 0)),
            pl.BlockSpec((1, 1), lambda i: (0, 0)),
        ],
        out_specs=pl.BlockSpec((1, 1), lambda i: (0, 0)),
        out_shape=jax.ShapeDtypeStruct((1, 1), jnp.float32),
        scratch_shapes=[pltpu.VMEM((1, _HID), jnp.float32)],
    )(agg2, deg_i, W2a, W2b, b2, Wf, bf)


def kernel(x, edge_index, W1, b1, W2, b2, Wf, bf):
    ei = edge_index.astype(jnp.int32)
    pad_fill = _N + jnp.arange(_EPAD - _E, dtype=jnp.int32) % (_NPAD - _N)
    ei = jnp.concatenate(
        [ei, jnp.stack([pad_fill, pad_fill])], axis=1)
    src2 = ei[0].reshape(_ECHUNKS, _K)
    dst2 = ei[1].reshape(_ECHUNKS, _K)
    x_pad = jnp.pad(x, ((0, _NPAD - _N), (0, 0)))

    deg = _deg_call(src2, dst2)
    deg_o = deg[0].reshape(_NPAD, 1)
    deg_i = deg[1].reshape(_NPAD, 1)

    xs = _scale_call(x_pad, deg_o)
    parts = _agg_edge_split_call(src2, dst2, xs)
    h1s = _layer1_call(parts, deg_i, deg_o, W1,
                       b1.reshape(1, _HID))
    agg2 = _agg_feat_split_call(src2, dst2, h1s)
    out = _layer2_call(agg2, deg_i, W2[:_DIN], W2[_DIN:],
                       b2.reshape(1, _HID), Wf, bf.reshape(1, 1))
    return out


# speedup vs baseline: 11.7643x; 1.0117x over previous
import functools

import jax
import jax.numpy as jnp
from jax import lax
from jax.experimental import pallas as pl
from jax.experimental.pallas import tpu as pltpu
from jax.experimental.pallas import tpu_sc as plsc

_N = 10000
_E = 320000
_DIN = 128
_HID = 256
_NPAD = 10240
_EPAD = 327680
_NSUB = 16
_NCORE = 2
_K = 128
_GRP = 16
_ECHUNKS = _EPAD // _K
_ROWS_PER_SUB = _NPAD // _NSUB
_ZROWS = 32
_BM = 1024
_GRID = _NPAD // _BM

_vec_mesh = plsc.VectorSubcoreMesh(core_axis_name="c", subcore_axis_name="s")


def _zero_fill_1d(buf, n):
    @pl.loop(0, n // 16)
    def _(i):
        buf[pl.ds(i * 16, 16)] = jnp.zeros((16,), jnp.float32)


def _zero_fill_2d(buf, rows, cols):
    @pl.loop(0, rows)
    def _(r):
        @pl.loop(0, cols // 16)
        def _(j):
            buf[r, pl.ds(j * 16, 16)] = jnp.zeros((16,), jnp.float32)


def _deg_call(src2, dst2):
    chunks = _ECHUNKS // _NSUB

    @functools.partial(
        pl.kernel,
        out_type=jax.ShapeDtypeStruct((_NCORE, _NPAD), jnp.float32),
        mesh=_vec_mesh,
        scratch_types=[
            pltpu.VMEM((chunks, _K), jnp.int32),
            pltpu.VMEM((_K,), jnp.float32),
            pltpu.VMEM((_ROWS_PER_SUB,), jnp.float32),
            pltpu.VMEM_SHARED((_NPAD,), jnp.float32),
            pltpu.SemaphoreType.DMA,
            pltpu.SemaphoreType.DMA,
        ],
    )
    def k(src_hbm, dst_hbm, out_hbm, idx_b, ones_b, zb, acc, isem, ssem):
        c = lax.axis_index("c")
        s = lax.axis_index("s")
        _zero_fill_1d(zb, _ROWS_PER_SUB)

        @pl.loop(0, _K // 16)
        def _(i):
            ones_b[pl.ds(i * 16, 16)] = jnp.ones((16,), jnp.float32)

        cbase = s * chunks

        @pl.when(c == 0)
        def _():
            pltpu.async_copy(src_hbm.at[pl.ds(cbase, chunks)], idx_b, isem)

        @pl.when(c == 1)
        def _():
            pltpu.async_copy(dst_hbm.at[pl.ds(cbase, chunks)], idx_b, isem)

        pltpu.sync_copy(zb, acc.at[pl.ds(s * _ROWS_PER_SUB, _ROWS_PER_SUB)])
        pltpu.make_async_copy(src_hbm.at[pl.ds(cbase, chunks)], idx_b,
                              isem).wait()
        plsc.subcore_barrier()

        @pl.loop(0, 8)
        def _(t):
            pltpu.async_copy(ones_b, acc.at[idx_b.at[t]], ssem, add=True)

        @pl.loop(8, chunks)
        def _(t):
            pltpu.make_async_copy(ones_b, acc.at[idx_b.at[t - 8]],
                                  ssem).wait()
            pltpu.async_copy(ones_b, acc.at[idx_b.at[t]], ssem, add=True)

        @pl.loop(chunks - 8, chunks)
        def _(t):
            pltpu.make_async_copy(ones_b, acc.at[idx_b.at[t]], ssem).wait()

        plsc.subcore_barrier()
        pltpu.sync_copy(
            acc.at[pl.ds(s * _ROWS_PER_SUB, _ROWS_PER_SUB)],
            out_hbm.at[c].at[pl.ds(s * _ROWS_PER_SUB, _ROWS_PER_SUB)],
        )

    return k(src2, dst2)


def _make_agg(edge_split):
    chunks = _ECHUNKS // (_NSUB * (_NCORE if edge_split else 1))
    grp = 40 if edge_split else 2 * _GRP
    ngroups = chunks // grp

    @functools.partial(
        pl.kernel,
        out_type=jax.ShapeDtypeStruct((_NCORE, _NPAD, _DIN), jnp.float32),
        mesh=_vec_mesh,
        scratch_types=[
            pltpu.VMEM((grp, _K), jnp.int32),
            pltpu.VMEM((grp, _K), jnp.int32),
            pltpu.VMEM((_K, _DIN), jnp.float32),
            pltpu.VMEM((_K, _DIN), jnp.float32),
            pltpu.VMEM((_ZROWS, _DIN), jnp.float32),
            pltpu.VMEM_SHARED((_NPAD, _DIN), jnp.float32),
            pltpu.SemaphoreType.DMA,
            pltpu.SemaphoreType.DMA,
            pltpu.SemaphoreType.DMA,
            pltpu.SemaphoreType.DMA,
            pltpu.SemaphoreType.DMA,
        ],
    )
    def k(src_hbm, dst_hbm, tab_hbm, out_hbm, src_b, dst_b, rows0, rows1,
          zb, acc, isem0, isem1, gsem0, gsem1, ssem0):
        c = lax.axis_index("c")
        s = lax.axis_index("s")
        rows = (rows0, rows1)
        gsem = (gsem0, gsem1)
        ssem = (isem1, ssem0)

        if edge_split:
            cbase = (c * _NSUB + s) * chunks
            tab = tab_hbm
        else:
            cbase = s * chunks
            tab = tab_hbm.at[c]

        _zero_fill_2d(zb, _ZROWS, _DIN)

        @pl.loop(0, _ROWS_PER_SUB // _ZROWS)
        def _(i):
            pltpu.sync_copy(
                zb, acc.at[pl.ds(s * _ROWS_PER_SUB + i * _ZROWS, _ZROWS)])

        plsc.subcore_barrier()

        def g_start(t, p):
            pltpu.async_copy(tab.at[src_b.at[t]], rows[p], gsem[p])

        def g_wait(t, p):
            pltpu.make_async_copy(tab.at[src_b.at[t]], rows[p],
                                  gsem[p]).wait()

        def s_start(t, p):
            pltpu.async_copy(rows[p], acc.at[dst_b.at[t]], ssem[p], add=True)

        def s_wait(t, p):
            pltpu.make_async_copy(rows[p], acc.at[dst_b.at[t]],
                                  ssem[p]).wait()

        @pl.loop(0, ngroups)
        def _(g):
            gb = cbase + g * grp
            pltpu.async_copy(src_hbm.at[pl.ds(gb, grp)], src_b, isem0)
            pltpu.async_copy(dst_hbm.at[pl.ds(gb, grp)], dst_b, gsem0)
            pltpu.make_async_copy(src_hbm.at[pl.ds(gb, grp)], src_b,
                                  isem0).wait()
            pltpu.make_async_copy(dst_hbm.at[pl.ds(gb, grp)], dst_b,
                                  gsem0).wait()

            g_start(0, 0)
            g_start(1, 1)
            g_wait(0, 0)
            s_start(0, 0)

            @pl.loop(1, grp // 2)
            def _(j):
                for b in range(2):
                    t = 2 * j + b
                    p = b
                    q = 1 - b
                    s_wait(t - 2, p)
                    g_start(t, p)
                    g_wait(t - 1, q)
                    s_start(t - 1, q)

            s_wait(grp - 2, 0)
            g_wait(grp - 1, 1)
            s_start(grp - 1, 1)
            s_wait(grp - 1, 1)

        plsc.subcore_barrier()
        pltpu.sync_copy(
            acc.at[pl.ds(s * _ROWS_PER_SUB, _ROWS_PER_SUB)],
            out_hbm.at[c].at[pl.ds(s * _ROWS_PER_SUB, _ROWS_PER_SUB)],
        )

    return k


_agg_edge_split_call = _make_agg(True)
_agg_feat_split_call = _make_agg(False)


def _scale_body(x_ref, dego_ref, o_ref):
    so = lax.rsqrt(jnp.maximum(dego_ref[...], 1.0))
    o_ref[...] = x_ref[...] * so


def _scale_call(x_pad, deg_o):
    return pl.pallas_call(
        _scale_body,
        grid=(_GRID,),
        in_specs=[
            pl.BlockSpec((_BM, _DIN), lambda i: (i, 0)),
            pl.BlockSpec((_BM, 1), lambda i: (i, 0)),
        ],
        out_specs=pl.BlockSpec((_BM, _DIN), lambda i: (i, 0)),
        out_shape=jax.ShapeDtypeStruct((_NPAD, _DIN), jnp.float32),
    )(x_pad, deg_o)


def _layer1_body(p_ref, degi_ref, dego_ref, w1_ref, b1_ref, o_ref):
    si = lax.rsqrt(jnp.maximum(degi_ref[...], 1.0))
    so = lax.rsqrt(jnp.maximum(dego_ref[...], 1.0))
    agg = (p_ref[0] + p_ref[1]) * si
    z = jnp.dot(agg, w1_ref[...], preferred_element_type=jnp.float32)
    h = jnp.maximum(z + b1_ref[...], 0.0) * so
    o_ref[0] = h[:, :_DIN]
    o_ref[1] = h[:, _DIN:]


def _layer1_call(parts, deg_i, deg_o, W1, b1):
    return pl.pallas_call(
        _layer1_body,
        grid=(_GRID,),
        in_specs=[
            pl.BlockSpec((2, _BM, _DIN), lambda i: (0, i, 0)),
            pl.BlockSpec((_BM, 1), lambda i: (i, 0)),
            pl.BlockSpec((_BM, 1), lambda i: (i, 0)),
            pl.BlockSpec((_DIN, _HID), lambda i: (0, 0)),
            pl.BlockSpec((1, _HID), lambda i: (0, 0)),
        ],
        out_specs=pl.BlockSpec((2, _BM, _DIN), lambda i: (0, i, 0)),
        out_shape=jax.ShapeDtypeStruct((2, _NPAD, _DIN), jnp.float32),
    )(parts, deg_i, deg_o, W1, b1)


def _layer2_body(agg_ref, degi_ref, w2a_ref, w2b_ref, b2_ref, wf_ref, bf_ref,
                 o_ref, acc_ref):
    b = pl.program_id(0)
    si = lax.rsqrt(jnp.maximum(degi_ref[...], 1.0))
    a0 = agg_ref[0] * si
    a1 = agg_ref[1] * si
    z = (jnp.dot(a0, w2a_ref[...], preferred_element_type=jnp.float32)
         + jnp.dot(a1, w2b_ref[...], preferred_element_type=jnp.float32)
         + b2_ref[...])
    h = jnp.maximum(z, 0.0)
    row = b * _BM + lax.broadcasted_iota(jnp.int32, (_BM, 1), 0)
    h = jnp.where(row < _N, h, 0.0)
    colsum = jnp.sum(h, axis=0, keepdims=True)

    @pl.when(b == 0)
    def _():
        acc_ref[...] = colsum

    @pl.when(b > 0)
    def _():
        acc_ref[...] = acc_ref[...] + colsum

    @pl.when(b == _GRID - 1)
    def _():
        hg = acc_ref[...] * (1.0 / _N)
        logit = jnp.dot(hg, wf_ref[...],
                        preferred_element_type=jnp.float32) + bf_ref[...]
        o_ref[...] = 1.0 / (1.0 + jnp.exp(-logit))


def _layer2_call(agg2, deg_i, W2a, W2b, b2, Wf, bf):
    return pl.pallas_call(
        _layer2_body,
        grid=(_GRID,),
        in_specs=[
            pl.BlockSpec((2, _BM, _DIN), lambda i: (0, i, 0)),
            pl.BlockSpec((_BM, 1), lambda i: (i, 0)),
            pl.BlockSpec((_DIN, _HID), lambda i: (0, 0)),
            pl.BlockSpec((_DIN, _HID), lambda i: (0, 0)),
            pl.BlockSpec((1, _HID), lambda i: (0, 0)),
            pl.BlockSpec((_HID, 1), lambda i: (0, 0)),
            pl.BlockSpec((1, 1), lambda i: (0, 0)),
        ],
        out_specs=pl.BlockSpec((1, 1), lambda i: (0, 0)),
        out_shape=jax.ShapeDtypeStruct((1, 1), jnp.float32),
        scratch_shapes=[pltpu.VMEM((1, _HID), jnp.float32)],
    )(agg2, deg_i, W2a, W2b, b2, Wf, bf)


def kernel(x, edge_index, W1, b1, W2, b2, Wf, bf):
    ei = edge_index.astype(jnp.int32)
    pad_fill = _N + jnp.arange(_EPAD - _E, dtype=jnp.int32) % (_NPAD - _N)
    ei = jnp.concatenate(
        [ei, jnp.stack([pad_fill, pad_fill])], axis=1)
    src2 = ei[0].reshape(_ECHUNKS, _K)
    dst2 = ei[1].reshape(_ECHUNKS, _K)
    x_pad = jnp.pad(x, ((0, _NPAD - _N), (0, 0)))

    deg = _deg_call(src2, dst2)
    deg_o = deg[0].reshape(_NPAD, 1)
    deg_i = deg[1].reshape(_NPAD, 1)

    xs = _scale_call(x_pad, deg_o)
    parts = _agg_edge_split_call(src2, dst2, xs)
    h1s = _layer1_call(parts, deg_i, deg_o, W1,
                       b1.reshape(1, _HID))
    agg2 = _agg_feat_split_call(src2, dst2, h1s)
    out = _layer2_call(agg2, deg_i, W2[:_DIN], W2[_DIN:],
                       b2.reshape(1, _HID), Wf, bf.reshape(1, 1))
    return out
